# baseline (speedup 1.0000x reference)
import functools

import jax
import jax.numpy as jnp
from jax import lax
from jax.experimental import pallas as pl
from jax.experimental.pallas import tpu as pltpu
from jax.experimental.pallas import tpu_sc as plsc

VOCAB = 100000
EMBED_DIM = 64
BATCH = 4096
SEQ = 200

FOLD = 16
WROWS = VOCAB // FOLD
WCOLS = FOLD * EMBED_DIM
WBLK = 1280
NWBLK = 5

VPAD = VOCAB

NC, NS, L = 2, 16, 16
NW = NC * NS
BPW = BATCH // NW
NFULL = SEQ // L
TAIL = SEQ - NFULL * L


def _tv_body(table_ref, k_ref, b_ref, out_ref):
    t = table_ref[...]
    k = k_ref[...]
    v = jnp.dot(t, k, preferred_element_type=jnp.float32)
    out_ref[...] = v + b_ref[0, 0]


def _compute_tv(table2, K, b2d):
    return pl.pallas_call(
        _tv_body,
        grid=(NWBLK,),
        in_specs=[
            pl.BlockSpec((WBLK, WCOLS), lambda i: (i, 0)),
            pl.BlockSpec((WCOLS, FOLD), lambda i: (0, 0)),
            pl.BlockSpec((1, 1), lambda i: (0, 0)),
        ],
        out_specs=pl.BlockSpec((WBLK, FOLD), lambda i: (i, 0)),
        out_shape=jax.ShapeDtypeStruct((WROWS, FOLD), jnp.float32),
    )(table2, K, b2d)


def _sc_body(tv_hbm, x_hbm, out_hbm, tv_v, idx_v, out_v):
    wid = lax.axis_index("s") * NC + lax.axis_index("c")
    base = wid * BPW
    pltpu.sync_copy(tv_hbm, tv_v)

    lanes = lax.iota(jnp.int32, L)
    UNROLL = 4
    HALF = BPW // 2
    for h in range(2):
        pltpu.sync_copy(x_hbm.at[pl.ds(base + h * HALF, HALF)], idx_v)
        for g in range(HALF // L):
            rows = jnp.full((L,), g * L, jnp.int32) + lanes

            def jstep(j, carry):
                accs, jv = carry
                new = []
                for u in range(UNROLL):
                    ii = plsc.load_gather(idx_v, [rows, jv + u])
                    new.append(accs[u] + plsc.load_gather(tv_v, [ii]))
                return tuple(new), jv + UNROLL

            accs, _ = lax.fori_loop(
                0, SEQ // UNROLL, jstep,
                (tuple(jnp.zeros((L,), jnp.float32) for _ in range(UNROLL)),
                 jnp.zeros((L,), jnp.int32)),
            )
            acc = (accs[0] + accs[1]) + (accs[2] + accs[3])
            out_v[pl.ds(h * HALF + g * L, L)] = acc * (1.0 / SEQ)

    pltpu.sync_copy(out_v, out_hbm.at[pl.ds(base, BPW)])


@functools.partial(jax.jit, static_argnames=())
def _pool(tv, x):
    mesh = plsc.VectorSubcoreMesh(core_axis_name="c", subcore_axis_name="s")
    f = pl.kernel(
        _sc_body,
        out_type=jax.ShapeDtypeStruct((BATCH,), jnp.float32),
        mesh=mesh,
        scratch_types=[
            pltpu.VMEM((VPAD,), jnp.float32),
            pltpu.VMEM((BPW // 2, SEQ), jnp.int32),
            pltpu.VMEM((BPW,), jnp.float32),
        ],
        compiler_params=pltpu.CompilerParams(needs_layout_passes=False),
    )
    return f(tv, x)


def kernel(x, table, W, b):
    table2 = table.reshape(WROWS, WCOLS)
    K = jnp.kron(jnp.eye(FOLD, dtype=jnp.float32), W)
    tv = _compute_tv(table2, K, b.reshape(1, 1).astype(jnp.float32))
    tv = tv.reshape(VOCAB)
    return _pool(tv, x.astype(jnp.int32))


# speedup vs baseline: 1.0178x; 1.0178x over previous
import functools

import jax
import jax.numpy as jnp
from jax import lax
from jax.experimental import pallas as pl
from jax.experimental.pallas import tpu as pltpu
from jax.experimental.pallas import tpu_sc as plsc

VOCAB = 100000
EMBED_DIM = 64
BATCH = 4096
SEQ = 200

FOLD = 16
WROWS = VOCAB // FOLD
WCOLS = FOLD * EMBED_DIM
WBLK = 1280
NWBLK = 5

NC, NS, L = 2, 16, 16
NW = NC * NS
BPW = BATCH // NW
NFULL = SEQ // L
TAIL = SEQ - NFULL * L


def _tv_body(table_ref, kt_ref, b_ref, out_ref):
    t = table_ref[...]
    kt = kt_ref[...]
    v = lax.dot_general(kt, t, (((1,), (1,)), ((), ())),
                        preferred_element_type=jnp.float32)
    out_ref[...] = v + b_ref[0, 0]


def _compute_tv(table2, KT, b2d):
    return pl.pallas_call(
        _tv_body,
        grid=(NWBLK,),
        in_specs=[
            pl.BlockSpec((WBLK, WCOLS), lambda i: (i, 0)),
            pl.BlockSpec((FOLD, WCOLS), lambda i: (0, 0)),
            pl.BlockSpec((1, 1), lambda i: (0, 0)),
        ],
        out_specs=pl.BlockSpec((FOLD, WBLK), lambda i: (0, i)),
        out_shape=jax.ShapeDtypeStruct((FOLD, WROWS), jnp.float32),
    )(table2, KT, b2d)


def _sc_body(tv_hbm, x_hbm, out_hbm, tv_v, idx_v, out_v):
    wid = lax.axis_index("s") * NC + lax.axis_index("c")
    base = wid * BPW
    pltpu.sync_copy(tv_hbm, tv_v)

    lanes = lax.iota(jnp.int32, L)
    UNROLL = 4
    HALF = BPW // 2
    for h in range(2):
        pltpu.sync_copy(x_hbm.at[pl.ds(base + h * HALF, HALF)], idx_v)
        for g in range(HALF // L):
            rows = jnp.full((L,), g * L, jnp.int32) + lanes

            def jstep(j, carry):
                accs, jv = carry
                new = []
                for u in range(UNROLL):
                    ii = plsc.load_gather(idx_v, [rows, jv + u])
                    f = jnp.bitwise_and(ii, FOLD - 1)
                    r = lax.shift_right_logical(ii, 4)
                    new.append(accs[u] + plsc.load_gather(tv_v, [f, r]))
                return tuple(new), jv + UNROLL

            accs, _ = lax.fori_loop(
                0, SEQ // UNROLL, jstep,
                (tuple(jnp.zeros((L,), jnp.float32) for _ in range(UNROLL)),
                 jnp.zeros((L,), jnp.int32)),
            )
            acc = (accs[0] + accs[1]) + (accs[2] + accs[3])
            out_v[pl.ds(h * HALF + g * L, L)] = acc * (1.0 / SEQ)

    pltpu.sync_copy(out_v, out_hbm.at[pl.ds(base, BPW)])


@functools.partial(jax.jit, static_argnames=())
def _pool(tv, x):
    mesh = plsc.VectorSubcoreMesh(core_axis_name="c", subcore_axis_name="s")
    f = pl.kernel(
        _sc_body,
        out_type=jax.ShapeDtypeStruct((BATCH,), jnp.float32),
        mesh=mesh,
        scratch_types=[
            pltpu.VMEM((FOLD, WROWS), jnp.float32),
            pltpu.VMEM((BPW // 2, SEQ), jnp.int32),
            pltpu.VMEM((BPW,), jnp.float32),
        ],
        compiler_params=pltpu.CompilerParams(needs_layout_passes=False),
    )
    return f(tv, x)


def kernel(x, table, W, b):
    table2 = table.reshape(WROWS, WCOLS)
    KT = jnp.kron(jnp.eye(FOLD, dtype=jnp.float32), W.T)
    tv = _compute_tv(table2, KT, b.reshape(1, 1).astype(jnp.float32))
    return _pool(tv, x.astype(jnp.int32))


# speedup vs baseline: 1.0444x; 1.0262x over previous
import functools

import jax
import jax.numpy as jnp
from jax import lax
from jax.experimental import pallas as pl
from jax.experimental.pallas import tpu as pltpu
from jax.experimental.pallas import tpu_sc as plsc

VOCAB = 100000
EMBED_DIM = 64
BATCH = 4096
SEQ = 200

TBLK = 2048
NTBLK = 49

NC, NS, L = 2, 16, 16
NW = NC * NS
BPW = BATCH // NW
NQ = 4
QR = BPW // NQ


def _tv_body(table_ref, wt_ref, b_ref, out_ref):
    t = table_ref[...]
    wt = wt_ref[...]
    v = lax.dot_general(wt, t, (((1,), (1,)), ((), ())),
                        preferred_element_type=jnp.float32)
    out_ref[...] = v + b_ref[0, 0]


def _compute_tv(table, WT, b2d):
    return pl.pallas_call(
        _tv_body,
        grid=(NTBLK,),
        in_specs=[
            pl.BlockSpec((TBLK, EMBED_DIM), lambda i: (i, 0)),
            pl.BlockSpec((1, EMBED_DIM), lambda i: (0, 0)),
            pl.BlockSpec((1, 1), lambda i: (0, 0)),
        ],
        out_specs=pl.BlockSpec((1, TBLK), lambda i: (0, i)),
        out_shape=jax.ShapeDtypeStruct((1, VOCAB), jnp.float32),
    )(table, WT, b2d)


def _sc_body(tv_hbm, x_hbm, out_hbm, tv_v, idx0_v, idx1_v, out_v, sem0, sem1):
    wid = lax.axis_index("s") * NC + lax.axis_index("c")
    base = wid * BPW

    bufs = (idx0_v, idx1_v)
    sems = (sem0, sem1)

    def start(q):
        return pltpu.async_copy(
            x_hbm.at[pl.ds(base + q * QR, QR)], bufs[q % 2], sems[q % 2])

    handles = [None] * NQ
    handles[0] = start(0)
    pltpu.sync_copy(tv_hbm.at[0], tv_v)

    lanes = lax.iota(jnp.int32, L)
    UNROLL = 4
    for q in range(NQ):
        if q + 1 < NQ:
            handles[q + 1] = start(q + 1)
        handles[q].wait()
        buf = bufs[q % 2]
        for g in range(QR // L):
            rows = jnp.full((L,), g * L, jnp.int32) + lanes

            def jstep(j, carry):
                accs, jv = carry
                new = []
                for u in range(UNROLL):
                    ii = plsc.load_gather(buf, [rows, jv + u])
                    new.append(accs[u] + plsc.load_gather(tv_v, [ii]))
                return tuple(new), jv + UNROLL

            accs, _ = lax.fori_loop(
                0, SEQ // UNROLL, jstep,
                (tuple(jnp.zeros((L,), jnp.float32) for _ in range(UNROLL)),
                 jnp.zeros((L,), jnp.int32)),
            )
            acc = (accs[0] + accs[1]) + (accs[2] + accs[3])
            out_v[pl.ds(q * QR + g * L, L)] = acc * (1.0 / SEQ)

    pltpu.sync_copy(out_v, out_hbm.at[pl.ds(base, BPW)])


@functools.partial(jax.jit, static_argnames=())
def _pool(tv, x):
    mesh = plsc.VectorSubcoreMesh(core_axis_name="c", subcore_axis_name="s")
    f = pl.kernel(
        _sc_body,
        out_type=jax.ShapeDtypeStruct((BATCH,), jnp.float32),
        mesh=mesh,
        scratch_types=[
            pltpu.VMEM((VOCAB,), jnp.float32),
            pltpu.VMEM((QR, SEQ), jnp.int32),
            pltpu.VMEM((QR, SEQ), jnp.int32),
            pltpu.VMEM((BPW,), jnp.float32),
            pltpu.SemaphoreType.DMA,
            pltpu.SemaphoreType.DMA,
        ],
        compiler_params=pltpu.CompilerParams(needs_layout_passes=False),
    )
    return f(tv, x)


def kernel(x, table, W, b):
    WT = W.reshape(1, EMBED_DIM)
    tv = _compute_tv(table, WT, b.reshape(1, 1).astype(jnp.float32))
    return _pool(tv, x.astype(jnp.int32))


# speedup vs baseline: 1.0561x; 1.0112x over previous
import functools

import jax
import jax.numpy as jnp
from jax import lax
from jax.experimental import pallas as pl
from jax.experimental.pallas import tpu as pltpu
from jax.experimental.pallas import tpu_sc as plsc

VOCAB = 100000
EMBED_DIM = 64
BATCH = 4096
SEQ = 200

FOLD = 2
WROWS = VOCAB // FOLD
WCOLS = FOLD * EMBED_DIM
WBLK = 6400
NWBLK = 8

NC, NS, L = 2, 16, 16
NW = NC * NS
BPW = BATCH // NW
NQ = 4
QR = BPW // NQ


def _tv_body2(table_ref, kt_ref, b_ref, out0_ref, out1_ref):
    t = table_ref[...]
    kt = kt_ref[...]
    v = lax.dot_general(kt, t, (((1,), (1,)), ((), ())),
                        preferred_element_type=jnp.float32)
    v = v + b_ref[0, 0]
    out0_ref[...] = v[0:1, :]
    out1_ref[...] = v[1:2, :]


def _compute_tv(table2, KT, b2d):
    return pl.pallas_call(
        _tv_body2,
        grid=(NWBLK,),
        in_specs=[
            pl.BlockSpec((WBLK, WCOLS), lambda i: (i, 0)),
            pl.BlockSpec((FOLD, WCOLS), lambda i: (0, 0)),
            pl.BlockSpec((1, 1), lambda i: (0, 0)),
        ],
        out_specs=[
            pl.BlockSpec((1, WBLK), lambda i: (0, i)),
            pl.BlockSpec((1, WBLK), lambda i: (0, i)),
        ],
        out_shape=[
            jax.ShapeDtypeStruct((1, WROWS), jnp.float32),
            jax.ShapeDtypeStruct((1, WROWS), jnp.float32),
        ],
    )(table2, KT, b2d)


def _sc_body(tv0_hbm, tv1_hbm, x_hbm, out_hbm, tv_v, idx0_v, idx1_v, out_v,
             sem0, sem1):
    wid = lax.axis_index("s") * NC + lax.axis_index("c")
    base = wid * BPW

    bufs = (idx0_v, idx1_v)
    sems = (sem0, sem1)

    def start(q):
        return pltpu.async_copy(
            x_hbm.at[pl.ds(base + q * QR, QR)], bufs[q % 2], sems[q % 2])

    handles = [None] * NQ
    handles[0] = start(0)
    pltpu.sync_copy(tv0_hbm.at[0], tv_v.at[pl.ds(0, WROWS)])
    pltpu.sync_copy(tv1_hbm.at[0], tv_v.at[pl.ds(WROWS, WROWS)])

    lanes = lax.iota(jnp.int32, L)
    UNROLL = 4
    for q in range(NQ):
        if q + 1 < NQ:
            handles[q + 1] = start(q + 1)
        handles[q].wait()
        buf = bufs[q % 2]
        for g in range(QR // L):
            rows = jnp.full((L,), g * L, jnp.int32) + lanes

            def jstep(j, carry):
                accs, jv = carry
                new = []
                for u in range(UNROLL):
                    ii = plsc.load_gather(buf, [rows, jv + u])
                    pos = (lax.shift_right_logical(ii, 1)
                           + jnp.bitwise_and(ii, 1) * WROWS)
                    new.append(accs[u] + plsc.load_gather(tv_v, [pos]))
                return tuple(new), jv + UNROLL

            accs, _ = lax.fori_loop(
                0, SEQ // UNROLL, jstep,
                (tuple(jnp.zeros((L,), jnp.float32) for _ in range(UNROLL)),
                 jnp.zeros((L,), jnp.int32)),
            )
            acc = (accs[0] + accs[1]) + (accs[2] + accs[3])
            out_v[pl.ds(q * QR + g * L, L)] = acc * (1.0 / SEQ)

    pltpu.sync_copy(out_v, out_hbm.at[pl.ds(base, BPW)])


@functools.partial(jax.jit, static_argnames=())
def _pool(tv0, tv1, x):
    mesh = plsc.VectorSubcoreMesh(core_axis_name="c", subcore_axis_name="s")
    f = pl.kernel(
        _sc_body,
        out_type=jax.ShapeDtypeStruct((BATCH,), jnp.float32),
        mesh=mesh,
        scratch_types=[
            pltpu.VMEM((VOCAB,), jnp.float32),
            pltpu.VMEM((QR, SEQ), jnp.int32),
            pltpu.VMEM((QR, SEQ), jnp.int32),
            pltpu.VMEM((BPW,), jnp.float32),
            pltpu.SemaphoreType.DMA,
            pltpu.SemaphoreType.DMA,
        ],
        compiler_params=pltpu.CompilerParams(needs_layout_passes=False),
    )
    return f(tv0, tv1, x)


def kernel(x, table, W, b):
    table2 = table.reshape(WROWS, WCOLS)
    KT = jnp.kron(jnp.eye(FOLD, dtype=jnp.float32), W.reshape(1, EMBED_DIM))
    tv0, tv1 = _compute_tv(table2, KT, b.reshape(1, 1).astype(jnp.float32))
    return _pool(tv0, tv1, x.astype(jnp.int32))


# speedup vs baseline: 1.0841x; 1.0265x over previous
import functools

import jax
import jax.numpy as jnp
from jax import lax
from jax.experimental import pallas as pl
from jax.experimental.pallas import tpu as pltpu
from jax.experimental.pallas import tpu_sc as plsc

VOCAB = 100000
EMBED_DIM = 64
BATCH = 4096
SEQ = 200

FOLD = 16
WROWS = VOCAB // FOLD
WCOLS = FOLD * EMBED_DIM
WBLK = 1280
NWBLK = 5

NC, NS, L = 2, 16, 16
NW = NC * NS
BPW = BATCH // NW


def _tv_body(table_ref, k_ref, b_ref, out_ref):
    t = table_ref[...]
    k = k_ref[...]
    v = jnp.dot(t, k, preferred_element_type=jnp.float32)
    out_ref[...] = v + b_ref[0, 0]


def _compute_tv(table2, K, b2d):
    return pl.pallas_call(
        _tv_body,
        grid=(NWBLK,),
        in_specs=[
            pl.BlockSpec((WBLK, WCOLS), lambda i: (i, 0)),
            pl.BlockSpec((WCOLS, FOLD), lambda i: (0, 0)),
            pl.BlockSpec((1, 1), lambda i: (0, 0)),
        ],
        out_specs=pl.BlockSpec((WBLK, FOLD), lambda i: (i, 0)),
        out_shape=jax.ShapeDtypeStruct((WROWS, FOLD), jnp.float32),
    )(table2, K, b2d)


def _sc_body(tv_hbm, x_hbm, out_hbm, tv_v, idx_v, out_v):
    wid = lax.axis_index("s") * NC + lax.axis_index("c")
    base = wid * BPW
    pltpu.sync_copy(x_hbm.at[pl.ds(base * SEQ, BPW * SEQ)], idx_v)
    pltpu.sync_copy(tv_hbm, tv_v)

    lanes = lax.iota(jnp.int32, L)
    UNROLL = 4
    for g in range(BPW // L):
        pos0 = (jnp.full((L,), g * L, jnp.int32) + lanes) * SEQ

        def jstep(j, carry):
            accs, pos = carry
            new = []
            for u in range(UNROLL):
                ii = plsc.load_gather(idx_v, [pos + u])
                new.append(accs[u] + plsc.load_gather(tv_v, [ii]))
            return tuple(new), pos + UNROLL

        accs, _ = lax.fori_loop(
            0, SEQ // UNROLL, jstep,
            (tuple(jnp.zeros((L,), jnp.float32) for _ in range(UNROLL)), pos0),
        )
        acc = (accs[0] + accs[1]) + (accs[2] + accs[3])
        out_v[pl.ds(g * L, L)] = acc * (1.0 / SEQ)

    pltpu.sync_copy(out_v, out_hbm.at[pl.ds(base, BPW)])


@functools.partial(jax.jit, static_argnames=())
def _pool(tv, x):
    mesh = plsc.VectorSubcoreMesh(core_axis_name="c", subcore_axis_name="s")
    f = pl.kernel(
        _sc_body,
        out_type=jax.ShapeDtypeStruct((BATCH,), jnp.float32),
        mesh=mesh,
        scratch_types=[
            pltpu.VMEM((VOCAB,), jnp.float32),
            pltpu.VMEM((BPW * SEQ,), jnp.int32),
            pltpu.VMEM((BPW,), jnp.float32),
        ],
        compiler_params=pltpu.CompilerParams(needs_layout_passes=False),
    )
    return f(tv, x)


def kernel(x, table, W, b):
    table2 = table.reshape(WROWS, WCOLS)
    K = jnp.kron(jnp.eye(FOLD, dtype=jnp.float32), W)
    tv = _compute_tv(table2, K, b.reshape(1, 1).astype(jnp.float32))
    tv = tv.reshape(VOCAB)
    xf = x.reshape(BATCH * SEQ).astype(jnp.int32)
    return _pool(tv, xf)


# speedup vs baseline: 1.0856x; 1.0014x over previous
import functools

import jax
import jax.numpy as jnp
from jax import lax
from jax.experimental import pallas as pl
from jax.experimental.pallas import tpu as pltpu
from jax.experimental.pallas import tpu_sc as plsc

VOCAB = 100000
EMBED_DIM = 64
BATCH = 4096
SEQ = 200

FOLD = 16
WROWS = VOCAB // FOLD
WCOLS = FOLD * EMBED_DIM
WBLK = 1280
NWBLK = 5

NC, NS, L = 2, 16, 16
NW = NC * NS
BPW = BATCH // NW


def _tv_body(table_ref, k_ref, b_ref, out_ref):
    t = table_ref[...]
    k = k_ref[...]
    v = jnp.dot(t, k, preferred_element_type=jnp.float32)
    out_ref[...] = v + b_ref[0, 0]


def _compute_tv(table2, K, b2d):
    return pl.pallas_call(
        _tv_body,
        grid=(NWBLK,),
        in_specs=[
            pl.BlockSpec((WBLK, WCOLS), lambda i: (i, 0)),
            pl.BlockSpec((WCOLS, FOLD), lambda i: (0, 0)),
            pl.BlockSpec((1, 1), lambda i: (0, 0)),
        ],
        out_specs=pl.BlockSpec((WBLK, FOLD), lambda i: (i, 0)),
        out_shape=jax.ShapeDtypeStruct((WROWS, FOLD), jnp.float32),
    )(table2, K, b2d)


def _sc_body(tv_hbm, x_hbm, out_hbm, tv_v, idx_v, out_v):
    wid = lax.axis_index("s") * NC + lax.axis_index("c")
    base = wid * BPW
    pltpu.sync_copy(x_hbm.at[pl.ds(base * SEQ, BPW * SEQ)], idx_v)
    pltpu.sync_copy(tv_hbm, tv_v)

    lanes = lax.iota(jnp.int32, L)
    UNROLL = 8
    for g in range(BPW // L):
        pos0 = (jnp.full((L,), g * L, jnp.int32) + lanes) * SEQ

        def jstep(j, carry):
            accs, pos = carry
            new = []
            for u in range(UNROLL):
                ii = plsc.load_gather(idx_v, [pos + u])
                new.append(accs[u] + plsc.load_gather(tv_v, [ii]))
            return tuple(new), pos + UNROLL

        accs, _ = lax.fori_loop(
            0, SEQ // UNROLL, jstep,
            (tuple(jnp.zeros((L,), jnp.float32) for _ in range(UNROLL)), pos0),
        )
        acc = ((accs[0] + accs[1]) + (accs[2] + accs[3])
               + (accs[4] + accs[5]) + (accs[6] + accs[7]))
        out_v[pl.ds(g * L, L)] = acc * (1.0 / SEQ)

    pltpu.sync_copy(out_v, out_hbm.at[pl.ds(base, BPW)])


@functools.partial(jax.jit, static_argnames=())
def _pool(tv, x):
    mesh = plsc.VectorSubcoreMesh(core_axis_name="c", subcore_axis_name="s")
    f = pl.kernel(
        _sc_body,
        out_type=jax.ShapeDtypeStruct((BATCH,), jnp.float32),
        mesh=mesh,
        scratch_types=[
            pltpu.VMEM((VOCAB,), jnp.float32),
            pltpu.VMEM((BPW * SEQ,), jnp.int32),
            pltpu.VMEM((BPW,), jnp.float32),
        ],
        compiler_params=pltpu.CompilerParams(needs_layout_passes=False),
    )
    return f(tv, x)


def kernel(x, table, W, b):
    table2 = table.reshape(WROWS, WCOLS)
    K = jnp.kron(jnp.eye(FOLD, dtype=jnp.float32), W)
    tv = _compute_tv(table2, K, b.reshape(1, 1).astype(jnp.float32))
    tv = tv.reshape(VOCAB)
    xf = x.reshape(BATCH * SEQ).astype(jnp.int32)
    return _pool(tv, xf)
